# Initial kernel scaffold; baseline (speedup 1.0000x reference)
import jax
import jax.numpy as jnp
from jax.experimental import pallas as pl


def kernel(x, weight):
    raise NotImplementedError("write your pallas kernel here")


# speedup vs baseline: 2.9856x; 2.9856x over previous
import functools

import jax
import jax.numpy as jnp
from jax import lax
from jax.experimental import pallas as pl
from jax.experimental.pallas import tpu as pltpu
from jax.experimental.pallas import tpu_sc as plsc

EMB = 128
B_TOT = 4096 * 50

_info = plsc.get_sparse_core_info()
NC = _info.num_cores
NS = _info.num_subcores
NW = NC * NS
BPW = B_TOT // NW
CH = 128
NSTEP = BPW // CH

_mesh = plsc.VectorSubcoreMesh(core_axis_name="c", subcore_axis_name="s")


@functools.partial(
    pl.kernel,
    mesh=_mesh,
    out_type=jax.ShapeDtypeStruct((B_TOT, EMB), jnp.float32),
    scratch_types=[
        pltpu.VMEM((BPW,), jnp.int32),
        pltpu.VMEM((CH, EMB), jnp.float32),
        pltpu.SemaphoreType.DMA,
    ],
)
def _embed(x_hbm, w_hbm, out_hbm, idx_v, rows_v, gsem):
    wid = lax.axis_index("s") * NC + lax.axis_index("c")
    base = wid * BPW
    pltpu.sync_copy(x_hbm.at[pl.ds(base, BPW)], idx_v)

    def body(step, carry):
        off = pl.multiple_of(step * CH, CH)
        pltpu.async_copy(w_hbm.at[idx_v.at[pl.ds(off, CH)]], rows_v, gsem).wait()
        pltpu.sync_copy(rows_v, out_hbm.at[pl.ds(base + off, CH)])
        return carry

    lax.fori_loop(0, NSTEP, body, 0)


def kernel(x, weight):
    xf = x.reshape(-1).astype(jnp.int32)
    out = _embed(xf, weight)
    return out.reshape(x.shape + (EMB,))


# speedup vs baseline: 3.3119x; 1.1093x over previous
import functools

import jax
import jax.numpy as jnp
from jax import lax
from jax.experimental import pallas as pl
from jax.experimental.pallas import tpu as pltpu
from jax.experimental.pallas import tpu_sc as plsc

EMB = 128
B_TOT = 4096 * 50

_info = plsc.get_sparse_core_info()
NC = _info.num_cores
NS = _info.num_subcores
NW = NC * NS
BPW = B_TOT // NW
CH = 128
NSTEP = BPW // CH
NBUF = 5
NOUT = NSTEP // NBUF

_mesh = plsc.VectorSubcoreMesh(core_axis_name="c", subcore_axis_name="s")


@functools.partial(
    pl.kernel,
    mesh=_mesh,
    out_type=jax.ShapeDtypeStruct((B_TOT, EMB), jnp.float32),
    scratch_types=(
        [pltpu.VMEM((BPW,), jnp.int32)]
        + [pltpu.VMEM((CH, EMB), jnp.float32) for _ in range(NBUF)]
        + [pltpu.SemaphoreType.DMA for _ in range(2 * NBUF)]
    ),
)
def _embed(x_hbm, w_hbm, out_hbm, idx_v, *bufs_and_sems):
    rows = bufs_and_sems[:NBUF]
    gsem = bufs_and_sems[NBUF:2 * NBUF]
    ssem = bufs_and_sems[2 * NBUF:]

    wid = lax.axis_index("s") * NC + lax.axis_index("c")
    base = wid * BPW
    pltpu.sync_copy(x_hbm.at[pl.ds(base, BPW)], idx_v)

    def gather(g, b):
        off = pl.multiple_of(g * CH, CH)
        pltpu.async_copy(w_hbm.at[idx_v.at[pl.ds(off, CH)]], rows[b], gsem[b])

    def gather_wait(b):
        pltpu.make_async_copy(
            w_hbm.at[idx_v.at[pl.ds(0, CH)]], rows[b], gsem[b]).wait()

    def store(g, b):
        off = pl.multiple_of(g * CH, CH)
        pltpu.async_copy(rows[b], out_hbm.at[pl.ds(base + off, CH)], ssem[b])

    def store_wait(b):
        pltpu.make_async_copy(
            rows[b], out_hbm.at[pl.ds(base, CH)], ssem[b]).wait()

    for b in range(NBUF):
        gather(b, b)

    def body(it, carry):
        g0 = it * NBUF
        for b in range(NBUF):
            gather_wait(b)
            store(g0 + b, b)
        for b in range(NBUF):
            @pl.when(it < NOUT - 1)
            def _():
                store_wait(b)
                gather(g0 + NBUF + b, b)
        return carry

    lax.fori_loop(0, NOUT, body, 0)

    for b in range(NBUF):
        store_wait(b)


def kernel(x, weight):
    xf = x.reshape(-1).astype(jnp.int32)
    out = _embed(xf, weight)
    return out.reshape(x.shape + (EMB,))
